# baseline (speedup 1.0000x reference)
import functools

import jax
import jax.numpy as jnp
from jax.experimental import pallas as pl
from jax.experimental.pallas import tpu as pltpu

_BN_EPS = 1e-5


def _fused_gcn_kernel(hf_ref, w_ref, gamma_ref, beta_ref, a_top_ref,
                      a_bot_ref, out_ref, ht_scratch):
    @pl.when(pl.program_id(0) == 0)
    def _compute_ht():
        ht = jnp.dot(hf_ref[...], w_ref[...],
                     preferred_element_type=jnp.float32)
        mean = jnp.mean(ht, axis=0, keepdims=True)
        var = jnp.mean(jnp.square(ht - mean), axis=0, keepdims=True)
        inv = jax.lax.rsqrt(var + _BN_EPS)
        htn = (ht - mean) * inv * gamma_ref[...] + beta_ref[...]
        ht_scratch[...] = jnp.maximum(htn, 0.0)

    out_ref[0] = jnp.dot(a_top_ref[...], ht_scratch[...],
                         preferred_element_type=jnp.float32)
    out_ref[1] = jnp.dot(a_bot_ref[...], ht_scratch[...],
                         preferred_element_type=jnp.float32)


@functools.partial(jax.jit, static_argnames=("block_m", "interpret"))
def _gcn(H, A_normalized, W, bn_gamma, bn_beta, block_m=200, interpret=False):
    batch, n, in_dim = H.shape
    out_dim = W.shape[1]
    hf = H.reshape(batch * n, in_dim)
    gamma = bn_gamma.reshape(1, out_dim)
    beta = bn_beta.reshape(1, out_dim)

    half_blocks = (n // 2) // block_m
    grid = (half_blocks,)
    out = pl.pallas_call(
        _fused_gcn_kernel,
        grid=grid,
        in_specs=[
            pl.BlockSpec((batch * n, in_dim), lambda i: (0, 0)),
            pl.BlockSpec((in_dim, out_dim), lambda i: (0, 0)),
            pl.BlockSpec((1, out_dim), lambda i: (0, 0)),
            pl.BlockSpec((1, out_dim), lambda i: (0, 0)),
            pl.BlockSpec((block_m, n), lambda i: (i, 0)),
            pl.BlockSpec((block_m, n),
                         lambda i, _hb=half_blocks: (i + _hb, 0)),
        ],
        out_specs=pl.BlockSpec((2, block_m, out_dim), lambda i: (0, i, 0)),
        out_shape=jax.ShapeDtypeStruct((2, n // 2, out_dim), jnp.float32),
        scratch_shapes=[pltpu.VMEM((batch * n, out_dim), jnp.float32)],
        interpret=interpret,
    )(hf, W, gamma, beta, A_normalized, A_normalized)
    return out.reshape(batch, n, out_dim)


def kernel(H, A_normalized, W, bn_gamma, bn_beta):
    return _gcn(H, A_normalized, W, bn_gamma, bn_beta, block_m=200)


# speedup vs baseline: 1.0047x; 1.0047x over previous
import functools

import jax
import jax.numpy as jnp
from jax.experimental import pallas as pl
from jax.experimental.pallas import tpu as pltpu

_BN_EPS = 1e-5
_NBUF = 3


def _make_kernel(n, block_m):
    nblk = n // block_m

    def _kernel(hf_ref, w_ref, gamma_ref, beta_ref, a_hbm, out_ref,
                ht_scratch, abuf, sems):
        def copy(i):
            return pltpu.make_async_copy(
                a_hbm.at[pl.ds(i * block_m, block_m), :],
                abuf.at[i % _NBUF],
                sems.at[i % _NBUF])

        for i in range(min(_NBUF, nblk)):
            copy(i).start()

        ht = jnp.dot(hf_ref[...], w_ref[...],
                     preferred_element_type=jnp.float32)
        mean = jnp.mean(ht, axis=0, keepdims=True)
        var = jnp.mean(jnp.square(ht - mean), axis=0, keepdims=True)
        inv = jax.lax.rsqrt(var + _BN_EPS)
        htn = (ht - mean) * inv * gamma_ref[...] + beta_ref[...]
        ht_scratch[...] = jnp.maximum(htn, 0.0)

        for i in range(nblk):
            copy(i).wait()
            out_ref[pl.ds(i * block_m, block_m), :] = jnp.dot(
                abuf[i % _NBUF], ht_scratch[...],
                preferred_element_type=jnp.float32)
            if i + _NBUF < nblk:
                copy(i + _NBUF).start()

    return _kernel


@functools.partial(jax.jit, static_argnames=("block_m", "interpret"))
def _gcn(H, A_normalized, W, bn_gamma, bn_beta, block_m=400, interpret=False):
    batch, n, in_dim = H.shape
    out_dim = W.shape[1]
    hf = H.reshape(batch * n, in_dim)
    gamma = bn_gamma.reshape(1, out_dim)
    beta = bn_beta.reshape(1, out_dim)

    out = pl.pallas_call(
        _make_kernel(n, block_m),
        in_specs=[
            pl.BlockSpec((batch * n, in_dim), lambda: (0, 0)),
            pl.BlockSpec((in_dim, out_dim), lambda: (0, 0)),
            pl.BlockSpec((1, out_dim), lambda: (0, 0)),
            pl.BlockSpec((1, out_dim), lambda: (0, 0)),
            pl.BlockSpec(memory_space=pltpu.MemorySpace.HBM),
        ],
        out_specs=pl.BlockSpec((n, out_dim), lambda: (0, 0)),
        out_shape=jax.ShapeDtypeStruct((n, out_dim), jnp.float32),
        scratch_shapes=[
            pltpu.VMEM((batch * n, out_dim), jnp.float32),
            pltpu.VMEM((_NBUF, block_m, n), jnp.float32),
            pltpu.SemaphoreType.DMA((_NBUF,)),
        ],
        compiler_params=pltpu.CompilerParams(
            vmem_limit_bytes=64 * 1024 * 1024),
        interpret=interpret,
    )(hf, W, gamma, beta, A_normalized)
    return out.reshape(batch, n, out_dim)


def kernel(H, A_normalized, W, bn_gamma, bn_beta):
    return _gcn(H, A_normalized, W, bn_gamma, bn_beta, block_m=400)


# speedup vs baseline: 1.0180x; 1.0132x over previous
import functools

import jax
import jax.numpy as jnp
from jax.experimental import pallas as pl
from jax.experimental.pallas import tpu as pltpu

_BN_EPS = 1e-5


def _fused_gcn_kernel(hf_ref, w_ref, gamma_ref, beta_ref, a_ref, out_ref,
                      ht_scratch):
    @pl.when(pl.program_id(0) == 0)
    def _compute_ht():
        ht = jnp.dot(hf_ref[...], w_ref[...],
                     preferred_element_type=jnp.float32)
        mean = jnp.mean(ht, axis=0, keepdims=True)
        var = jnp.mean(jnp.square(ht - mean), axis=0, keepdims=True)
        inv = jax.lax.rsqrt(var + _BN_EPS)
        htn = (ht - mean) * inv * gamma_ref[...] + beta_ref[...]
        ht_scratch[...] = jnp.maximum(htn, 0.0)

    out_ref[...] = jnp.dot(a_ref[...], ht_scratch[...],
                           preferred_element_type=jnp.float32)


@functools.partial(jax.jit, static_argnames=("block_m", "interpret"))
def _gcn(H, A_normalized, W, bn_gamma, bn_beta, block_m=400, interpret=False):
    batch, n, in_dim = H.shape
    out_dim = W.shape[1]
    hf = H.reshape(batch * n, in_dim)
    gamma = bn_gamma.reshape(1, out_dim)
    beta = bn_beta.reshape(1, out_dim)

    grid = (n // block_m,)
    out = pl.pallas_call(
        _fused_gcn_kernel,
        grid=grid,
        in_specs=[
            pl.BlockSpec((batch * n, in_dim), lambda i: (0, 0)),
            pl.BlockSpec((in_dim, out_dim), lambda i: (0, 0)),
            pl.BlockSpec((1, out_dim), lambda i: (0, 0)),
            pl.BlockSpec((1, out_dim), lambda i: (0, 0)),
            pl.BlockSpec((block_m, n), lambda i: (i, 0)),
        ],
        out_specs=pl.BlockSpec((block_m, out_dim), lambda i: (i, 0)),
        out_shape=jax.ShapeDtypeStruct((n, out_dim), jnp.float32),
        scratch_shapes=[pltpu.VMEM((batch * n, out_dim), jnp.float32)],
        compiler_params=pltpu.CompilerParams(
            vmem_limit_bytes=64 * 1024 * 1024),
        interpret=interpret,
    )(hf, W, gamma, beta, A_normalized)
    return out.reshape(batch, n, out_dim)


def kernel(H, A_normalized, W, bn_gamma, bn_beta):
    return _gcn(H, A_normalized, W, bn_gamma, bn_beta, block_m=400)
